# baseline (speedup 1.0000x reference)
import functools

import jax
import jax.numpy as jnp
from jax import lax
from jax.experimental import pallas as pl
from jax.experimental.pallas import tpu as pltpu
from jax.experimental.pallas import tpu_sc as plsc

N = 10000
E = 320000
C = 128

NUM_CORES = 2
NUM_SUBCORES = 16
NUM_TILES = NUM_CORES * NUM_SUBCORES
CHUNK = 80
NUM_GROUPS = 5
CPG = 25
EDGES_PER_TILE = NUM_GROUPS * CPG * CHUNK
ACC_ROWS = N
STRIPE = 624
TAIL = N - STRIPE * NUM_SUBCORES


def _sc_agg_body(vals_hbm, src_hbm, dst_hbm, zeros_hbm, out_hbm,
                 src_v0, dst_v0, rows0, rows1, rows2, acc_sh,
                 gsem0, gsem1, gsem2, ssem0, ssem1, ssem2):
    cid = lax.axis_index("c")
    sid = lax.axis_index("s")
    wid = sid * NUM_CORES + cid

    r0 = sid * STRIPE

    def _stripe_copy(src_fn, dst_fn):
        pltpu.sync_copy(src_fn(pl.ds(r0, STRIPE)), dst_fn(pl.ds(r0, STRIPE)))

        @pl.when(sid == NUM_SUBCORES - 1)
        def _():
            pltpu.sync_copy(src_fn(pl.ds(STRIPE * NUM_SUBCORES, TAIL)),
                            dst_fn(pl.ds(STRIPE * NUM_SUBCORES, TAIL)))

    @pl.when(cid == 0)
    def _():
        _stripe_copy(lambda s: vals_hbm.at[s], lambda s: acc_sh.at[s])

    @pl.when(cid != 0)
    def _():
        _stripe_copy(lambda s: zeros_hbm.at[s], lambda s: acc_sh.at[s])

    plsc.subcore_barrier()

    sv, dv = src_v0, dst_v0
    bufs = ((rows0, gsem0, ssem0), (rows1, gsem1, ssem1),
            (rows2, gsem2, ssem2))

    def gath(j, b):
        pltpu.async_copy(vals_hbm.at[sv.at[j]], b[0], b[1])

    def wait_g(j, b):
        pltpu.make_async_copy(vals_hbm.at[sv.at[j]], b[0], b[1]).wait()

    def scat(j, b):
        pltpu.async_copy(b[0], acc_sh.at[dv.at[j]], b[2], add=True)

    def wait_s(j, b):
        pltpu.make_async_copy(b[0], acc_sh.at[dv.at[j]], b[2]).wait()

    for g in range(NUM_GROUPS):
        pltpu.sync_copy(src_hbm.at[wid].at[g], sv)
        pltpu.sync_copy(dst_hbm.at[wid].at[g], dv)

        for i in range(3):
            gath(i, bufs[i])

        @pl.loop(0, (CPG - 4) // 3)
        def _(k):
            t = 3 * k
            for i in range(3):
                wait_g(t + i, bufs[i])
                scat(t + i, bufs[i])
            for i in range(3):
                wait_s(t + i, bufs[i])
                gath(t + 3 + i, bufs[i])

        t0 = 3 * ((CPG - 4) // 3)
        for i in range(3):
            wait_g(t0 + i, bufs[i])
            scat(t0 + i, bufs[i])
        wait_s(t0, bufs[0])
        gath(CPG - 1, bufs[0])
        wait_g(CPG - 1, bufs[0])
        scat(CPG - 1, bufs[0])
        wait_s(CPG - 1, bufs[0])
        wait_s(t0 + 1, bufs[1])
        wait_s(t0 + 2, bufs[2])

    plsc.subcore_barrier()
    _stripe_copy(lambda s: acc_sh.at[s], lambda s: out_hbm.at[cid].at[s])


def _sc_agg(values, src3, dst3, zeros):
    mesh = plsc.VectorSubcoreMesh(core_axis_name="c", subcore_axis_name="s")
    k = pl.kernel(
        _sc_agg_body,
        out_type=jax.ShapeDtypeStruct((NUM_CORES, N, C), jnp.float32),
        mesh=mesh,
        scratch_types=[
            pltpu.VMEM((CPG, CHUNK), jnp.int32),
            pltpu.VMEM((CPG, CHUNK), jnp.int32),
            pltpu.VMEM((CHUNK, C), jnp.float32),
            pltpu.VMEM((CHUNK, C), jnp.float32),
            pltpu.VMEM((CHUNK, C), jnp.float32),
            pltpu.VMEM_SHARED((ACC_ROWS, C), jnp.float32),
        ] + [pltpu.SemaphoreType.DMA] * 6,
    )
    return k(values, src3, dst3, zeros)


def _bn_cols(h, g, b):
    m = jnp.mean(h, axis=0, keepdims=True)
    v = jnp.mean((h - m) * (h - m), axis=0, keepdims=True)
    return (h - m) * lax.rsqrt(v + 1e-5) * g + b


def _bn_in_body(x_ref, g_ref, b_ref, o_ref):
    o_ref[...] = _bn_cols(x_ref[...], g_ref[...], b_ref[...])


def _mlp_relu_body(p_ref, w1_ref, b1_ref, w2_ref, b2_ref, o_ref):
    p = p_ref[...]
    h = p[0] + p[1]
    t = jnp.maximum(
        jnp.dot(h, w1_ref[...], preferred_element_type=jnp.float32)
        + b1_ref[...], 0.0)
    o = jnp.dot(t, w2_ref[...], preferred_element_type=jnp.float32) + b2_ref[...]
    o_ref[...] = jnp.maximum(o, 0.0)


def _mlp_bn_tanh_body(p_ref, w1_ref, b1_ref, w2_ref, b2_ref, g_ref, bb_ref,
                      o_ref):
    p = p_ref[...]
    h = p[0] + p[1]
    t = jnp.maximum(
        jnp.dot(h, w1_ref[...], preferred_element_type=jnp.float32)
        + b1_ref[...], 0.0)
    o = jnp.dot(t, w2_ref[...], preferred_element_type=jnp.float32) + b2_ref[...]
    o_ref[...] = jnp.tanh(_bn_cols(o, g_ref[...], bb_ref[...]))


_f32 = functools.partial(jax.ShapeDtypeStruct, dtype=jnp.float32)


def _bn_in(x, g, b):
    return pl.pallas_call(_bn_in_body, out_shape=_f32((N, C)))(
        x, g.reshape(1, C), b.reshape(1, C))


def _mlp_relu(parts, w1, b1, w2, b2):
    return pl.pallas_call(_mlp_relu_body, out_shape=_f32((N, C)))(
        parts, w1, b1.reshape(1, C), w2, b2.reshape(1, C))


def _mlp_bn_tanh(parts, w1, b1, w2, b2, g, bb):
    return pl.pallas_call(_mlp_bn_tanh_body, out_shape=_f32((N, C)))(
        parts, w1, b1.reshape(1, C), w2, b2.reshape(1, C),
        g.reshape(1, C), bb.reshape(1, C))


@jax.jit
def kernel(x, edge_index_p, edge_index_s, edge_index_v, in_gamma, in_beta,
           W11, b11, W12, b12, W21, b21, W22, b22, bn_gamma, bn_beta):
    zeros = jnp.zeros((N, C), jnp.float32)
    xn = _bn_in(x, in_gamma, in_beta)
    outs = []
    for i, ei in enumerate((edge_index_p, edge_index_s, edge_index_v)):
        src3 = ei[0].reshape(NUM_TILES, NUM_GROUPS, CPG, CHUNK)
        dst3 = ei[1].reshape(NUM_TILES, NUM_GROUPS, CPG, CHUNK)
        parts1 = _sc_agg(xn, src3, dst3, zeros)
        h1 = _mlp_relu(parts1, W11[i], b11[i], W12[i], b12[i])
        parts2 = _sc_agg(h1, src3, dst3, zeros)
        outs.append(_mlp_bn_tanh(parts2, W21[i], b21[i], W22[i], b22[i],
                                 bn_gamma[i], bn_beta[i]))
    return tuple(outs)


# speedup vs baseline: 1.0981x; 1.0981x over previous
import functools

import jax
import jax.numpy as jnp
from jax import lax
from jax.experimental import pallas as pl
from jax.experimental.pallas import tpu as pltpu
from jax.experimental.pallas import tpu_sc as plsc

N = 10000
E = 320000
C = 128

NUM_CORES = 2
NUM_SUBCORES = 16
NUM_TILES = NUM_CORES * NUM_SUBCORES
CHUNK = 80
NUM_GROUPS = 5
CPG = 25
EDGES_PER_TILE = NUM_GROUPS * CPG * CHUNK
ACC_ROWS = N
STRIPE = 624
TAIL = N - STRIPE * NUM_SUBCORES


def _sc_agg_body(vals_hbm, src_hbm, dst_hbm, zeros_hbm, out_hbm,
                 src_v0, dst_v0, src_v1, dst_v1, rows0, rows1, acc_sh,
                 gsem0, gsem1, is0, id0, is1, id1):
    cid = lax.axis_index("c")
    sid = lax.axis_index("s")
    wid = sid * NUM_CORES + cid

    r0 = sid * STRIPE

    def _stripe_copy(src_fn, dst_fn):
        pltpu.sync_copy(src_fn(pl.ds(r0, STRIPE)), dst_fn(pl.ds(r0, STRIPE)))

        @pl.when(sid == NUM_SUBCORES - 1)
        def _():
            pltpu.sync_copy(src_fn(pl.ds(STRIPE * NUM_SUBCORES, TAIL)),
                            dst_fn(pl.ds(STRIPE * NUM_SUBCORES, TAIL)))

    @pl.when(cid == 0)
    def _():
        _stripe_copy(lambda s: vals_hbm.at[s], lambda s: acc_sh.at[s])

    @pl.when(cid != 0)
    def _():
        _stripe_copy(lambda s: zeros_hbm.at[s], lambda s: acc_sh.at[s])

    pltpu.async_copy(src_hbm.at[wid].at[0], src_v0, is0)
    pltpu.async_copy(dst_hbm.at[wid].at[0], dst_v0, id0)
    plsc.subcore_barrier()

    pairs = ((src_v0, dst_v0, is0, id0), (src_v1, dst_v1, is1, id1))
    for g in range(NUM_GROUPS):
        sv, dv, isem, idsem = pairs[g % 2]
        if g + 1 < NUM_GROUPS:
            nsv, ndv, nisem, nidsem = pairs[(g + 1) % 2]
            pltpu.async_copy(src_hbm.at[wid].at[g + 1], nsv, nisem)
            pltpu.async_copy(dst_hbm.at[wid].at[g + 1], ndv, nidsem)
        pltpu.make_async_copy(src_hbm.at[wid].at[g], sv, isem).wait()
        pltpu.make_async_copy(dst_hbm.at[wid].at[g], dv, idsem).wait()

        def gath(j, buf, sem):
            pltpu.async_copy(vals_hbm.at[sv.at[j]], buf, sem)

        def wait_g(j, buf, sem):
            pltpu.make_async_copy(vals_hbm.at[sv.at[j]], buf, sem).wait()

        def scat(j, buf):
            pltpu.sync_copy(buf, acc_sh.at[dv.at[j]], add=True)

        gath(0, rows0, gsem0)

        @pl.loop(0, CPG - 1, step=2)
        def _(t):
            gath(t + 1, rows1, gsem1)
            wait_g(t, rows0, gsem0)
            scat(t, rows0)
            gath(t + 2, rows0, gsem0)
            wait_g(t + 1, rows1, gsem1)
            scat(t + 1, rows1)

        wait_g(CPG - 1, rows0, gsem0)
        scat(CPG - 1, rows0)

    plsc.subcore_barrier()
    _stripe_copy(lambda s: acc_sh.at[s], lambda s: out_hbm.at[cid].at[s])


def _sc_agg(values, src3, dst3, zeros):
    mesh = plsc.VectorSubcoreMesh(core_axis_name="c", subcore_axis_name="s")
    k = pl.kernel(
        _sc_agg_body,
        out_type=jax.ShapeDtypeStruct((NUM_CORES, N, C), jnp.float32),
        mesh=mesh,
        scratch_types=[
            pltpu.VMEM((CPG, CHUNK), jnp.int32),
            pltpu.VMEM((CPG, CHUNK), jnp.int32),
            pltpu.VMEM((CPG, CHUNK), jnp.int32),
            pltpu.VMEM((CPG, CHUNK), jnp.int32),
            pltpu.VMEM((CHUNK, C), jnp.float32),
            pltpu.VMEM((CHUNK, C), jnp.float32),
            pltpu.VMEM_SHARED((ACC_ROWS, C), jnp.float32),
        ] + [pltpu.SemaphoreType.DMA] * 6,
    )
    return k(values, src3, dst3, zeros)


def _bn_cols(h, g, b):
    m = jnp.mean(h, axis=0, keepdims=True)
    v = jnp.mean((h - m) * (h - m), axis=0, keepdims=True)
    return (h - m) * lax.rsqrt(v + 1e-5) * g + b


def _bn_in_body(x_ref, g_ref, b_ref, o_ref):
    o_ref[...] = _bn_cols(x_ref[...], g_ref[...], b_ref[...])


def _mlp_relu_body(p_ref, w1_ref, b1_ref, w2_ref, b2_ref, o_ref):
    p = p_ref[...]
    h = p[0] + p[1]
    t = jnp.maximum(
        jnp.dot(h, w1_ref[...], preferred_element_type=jnp.float32)
        + b1_ref[...], 0.0)
    o = jnp.dot(t, w2_ref[...], preferred_element_type=jnp.float32) + b2_ref[...]
    o_ref[...] = jnp.maximum(o, 0.0)


def _mlp_bn_tanh_body(p_ref, w1_ref, b1_ref, w2_ref, b2_ref, g_ref, bb_ref,
                      o_ref):
    p = p_ref[...]
    h = p[0] + p[1]
    t = jnp.maximum(
        jnp.dot(h, w1_ref[...], preferred_element_type=jnp.float32)
        + b1_ref[...], 0.0)
    o = jnp.dot(t, w2_ref[...], preferred_element_type=jnp.float32) + b2_ref[...]
    o_ref[...] = jnp.tanh(_bn_cols(o, g_ref[...], bb_ref[...]))


_f32 = functools.partial(jax.ShapeDtypeStruct, dtype=jnp.float32)


def _bn_in(x, g, b):
    return pl.pallas_call(_bn_in_body, out_shape=_f32((N, C)))(
        x, g.reshape(1, C), b.reshape(1, C))


def _mlp_relu(parts, w1, b1, w2, b2):
    return pl.pallas_call(_mlp_relu_body, out_shape=_f32((N, C)))(
        parts, w1, b1.reshape(1, C), w2, b2.reshape(1, C))


def _mlp_bn_tanh(parts, w1, b1, w2, b2, g, bb):
    return pl.pallas_call(_mlp_bn_tanh_body, out_shape=_f32((N, C)))(
        parts, w1, b1.reshape(1, C), w2, b2.reshape(1, C),
        g.reshape(1, C), bb.reshape(1, C))


@jax.jit
def kernel(x, edge_index_p, edge_index_s, edge_index_v, in_gamma, in_beta,
           W11, b11, W12, b12, W21, b21, W22, b22, bn_gamma, bn_beta):
    zeros = jnp.zeros((N, C), jnp.float32)
    xn = _bn_in(x, in_gamma, in_beta)
    outs = []
    for i, ei in enumerate((edge_index_p, edge_index_s, edge_index_v)):
        src3 = ei[0].reshape(NUM_TILES, NUM_GROUPS, CPG, CHUNK)
        dst3 = ei[1].reshape(NUM_TILES, NUM_GROUPS, CPG, CHUNK)
        parts1 = _sc_agg(xn, src3, dst3, zeros)
        h1 = _mlp_relu(parts1, W11[i], b11[i], W12[i], b12[i])
        parts2 = _sc_agg(h1, src3, dst3, zeros)
        outs.append(_mlp_bn_tanh(parts2, W21[i], b21[i], W22[i], b22[i],
                                 bn_gamma[i], bn_beta[i]))
    return tuple(outs)
